# baseline (speedup 1.0000x reference)
import functools

import jax
import jax.numpy as jnp
from jax import lax
from jax.experimental import pallas as pl
from jax.experimental.pallas import tpu as pltpu
from jax.experimental.pallas import tpu_sc as plsc

NC, NS = 2, 16
NW = NC * NS
LANES = 16
C = 16

_SC_PARAMS = pltpu.CompilerParams(
    needs_layout_passes=False, use_tc_tiling_on_sc=False)


def _unpack_row(rows, r, D):
    out = []
    for h in range(D // 32):
        half = rows[r, pl.ds(h * 32, 32)]
        a, b = plsc.unpack(half, format=plsc.PackFormat.INTERLEAVED)
        out += [a, b]
    return out


def _ctx_means(ctx_flat, in_bf, B, CTX, D):
    BW = B // NW
    NIT = BW // C
    NKC = D // LANES
    nctx = C * CTX
    mesh = plsc.VectorSubcoreMesh(core_axis_name="c", subcore_axis_name="s")

    @functools.partial(
        pl.kernel,
        out_type=jax.ShapeDtypeStruct((B * D,), jnp.float32),
        mesh=mesh,
        compiler_params=_SC_PARAMS,
        scratch_types=[
            pltpu.VMEM((BW * CTX,), jnp.int32),
            pltpu.VMEM((nctx, D), jnp.bfloat16),
            pltpu.VMEM((nctx, D), jnp.bfloat16),
            pltpu.VMEM((BW * D,), jnp.float32),
            pltpu.SemaphoreType.DMA,
            pltpu.SemaphoreType.DMA,
        ],
    )
    def mean_kernel(ctx_hbm, ine_hbm, mean_o_hbm,
                    ctx_idx, rows0, rows1, mean_buf, sem0, sem1):
        wid = lax.axis_index("s") * NC + lax.axis_index("c")
        pltpu.sync_copy(ctx_hbm.at[pl.ds(wid * BW * CTX, BW * CTX)], ctx_idx)

        def fire(i, buf, sem):
            ds = []
            for h in range(2):
                ds.append(pltpu.async_copy(
                    ine_hbm.at[ctx_idx.at[pl.ds(i * nctx + h * (nctx // 2),
                                                nctx // 2)]],
                    buf.at[pl.ds(h * (nctx // 2), nctx // 2)], sem))
            return ds

        def drain(i, buf, sem):
            for h in range(2):
                pltpu.make_async_copy(
                    ine_hbm.at[ctx_idx.at[pl.ds(i * nctx + h * (nctx // 2),
                                                nctx // 2)]],
                    buf.at[pl.ds(h * (nctx // 2), nctx // 2)], sem).wait()

        def compute(i, rows):
            for b in range(C):
                m = _unpack_row(rows, b * CTX, D)
                for c in range(1, CTX):
                    r = _unpack_row(rows, b * CTX + c, D)
                    m = [mk + rk for mk, rk in zip(m, r)]
                base = (i * C + b) * D
                for k in range(NKC):
                    mean_buf[pl.ds(base + k * LANES, LANES)] = (
                        m[k] * (1.0 / CTX))

        fire(0, rows0, sem0)

        def body(j, carry):
            i0 = 2 * j
            fire(i0 + 1, rows1, sem1)
            drain(i0, rows0, sem0)
            compute(i0, rows0)
            fire(jnp.minimum(i0 + 2, NIT - 1), rows0, sem0)
            drain(i0 + 1, rows1, sem1)
            compute(i0 + 1, rows1)
            return carry

        lax.fori_loop(0, NIT // 2, body, 0)
        drain(NIT - 1, rows0, sem0)
        pltpu.sync_copy(mean_buf, mean_o_hbm.at[pl.ds(wid * BW * D, BW * D)])

    return mean_kernel(ctx_flat, in_bf)


def _scores(cen, neg_flat, means, out_bf, B, NEG, D):
    BW = B // NW
    NIT = BW // C
    NKC = D // LANES
    nneg = C * NEG
    mesh = plsc.VectorSubcoreMesh(core_axis_name="c", subcore_axis_name="s")

    @functools.partial(
        pl.kernel,
        out_type=(
            jax.ShapeDtypeStruct((B,), jnp.float32),
            jax.ShapeDtypeStruct((B * NEG,), jnp.float32),
        ),
        mesh=mesh,
        compiler_params=_SC_PARAMS,
        scratch_types=[
            pltpu.VMEM((BW,), jnp.int32),
            pltpu.VMEM((BW * NEG,), jnp.int32),
            pltpu.VMEM((BW * D,), jnp.float32),
            pltpu.VMEM((C, D), jnp.bfloat16),
            pltpu.VMEM((nneg, D), jnp.bfloat16),
            pltpu.VMEM((C * LANES,), jnp.float32),
            pltpu.VMEM((nneg * LANES,), jnp.float32),
            pltpu.VMEM((BW,), jnp.float32),
            pltpu.VMEM((BW * NEG,), jnp.float32),
            pltpu.SemaphoreType.DMA,
        ],
    )
    def score_kernel(cen_hbm, neg_hbm, mean_hbm, oute_hbm,
                     pos_o_hbm, neg_o_hbm,
                     cen_idx, neg_idx, mean_v, pos_rows, neg_rows,
                     stage_pos, stage_neg, pos_buf, neg_buf, sem):
        wid = lax.axis_index("s") * NC + lax.axis_index("c")
        pltpu.sync_copy(cen_hbm.at[pl.ds(wid * BW, BW)], cen_idx)
        pltpu.sync_copy(neg_hbm.at[pl.ds(wid * BW * NEG, BW * NEG)], neg_idx)
        pltpu.sync_copy(mean_hbm.at[pl.ds(wid * BW * D, BW * D)], mean_v)

        def colsum(stage, r0):
            base = lax.iota(jnp.int32, 16) * LANES + (r0 * LANES)
            acc = plsc.load_gather(stage, [base])
            for c in range(1, LANES):
                acc = acc + plsc.load_gather(stage, [base + c])
            return acc

        def body(i, carry):
            dmas = [pltpu.async_copy(
                oute_hbm.at[cen_idx.at[pl.ds(i * C, C)]], pos_rows, sem)]
            for h in range(4):
                dmas.append(pltpu.async_copy(
                    oute_hbm.at[neg_idx.at[pl.ds(i * nneg + h * (nneg // 4),
                                                 nneg // 4)]],
                    neg_rows.at[pl.ds(h * (nneg // 4), nneg // 4)], sem))
            for d in dmas:
                d.wait()

            for b in range(C):
                base = (i * C + b) * D
                m = [mean_v[pl.ds(base + k * LANES, LANES)]
                     for k in range(NKC)]

                r = _unpack_row(pos_rows, b, D)
                p = m[0] * r[0]
                for k in range(1, NKC):
                    p = p + m[k] * r[k]
                stage_pos[pl.ds(b * LANES, LANES)] = p

                for n in range(NEG):
                    row = b * NEG + n
                    r = _unpack_row(neg_rows, row, D)
                    q = m[0] * r[0]
                    for k in range(1, NKC):
                        q = q + m[k] * r[k]
                    stage_neg[pl.ds(row * LANES, LANES)] = q

            pos_buf[pl.ds(i * C, C)] = colsum(stage_pos, 0)
            for g in range(nneg // LANES):
                neg_buf[pl.ds(i * nneg + g * LANES, LANES)] = (
                    colsum(stage_neg, g * LANES))
            return carry

        lax.fori_loop(0, NIT, body, 0)
        pltpu.sync_copy(pos_buf, pos_o_hbm.at[pl.ds(wid * BW, BW)])
        pltpu.sync_copy(neg_buf, neg_o_hbm.at[pl.ds(wid * BW * NEG, BW * NEG)])

    return score_kernel(cen, neg_flat, means, out_bf)


def _loss_from_scores(pos_score, neg_score_flat, B):
    pos2 = pos_score.reshape(-1, 128)
    neg2 = neg_score_flat.reshape(-1, 128)

    def body(p_ref, n_ref, o_ref):
        def neg_softplus(x):
            return jnp.minimum(x, 0.0) - jnp.log(1.0 + jnp.exp(-jnp.abs(x)))

        total = jnp.sum(neg_softplus(p_ref[...]))
        total = total + jnp.sum(neg_softplus(-n_ref[...]))
        o_ref[0, 0] = -total / B

    out = pl.pallas_call(
        body,
        out_shape=jax.ShapeDtypeStruct((1, 1), jnp.float32),
        out_specs=pl.BlockSpec(memory_space=pltpu.SMEM),
    )(pos2, neg2)
    return out[0, 0]


def kernel(context_words, center_word, neg_words, in_embed, out_embed):
    B, CTX = context_words.shape
    NEG = neg_words.shape[1]
    D = in_embed.shape[1]
    in_bf = in_embed.astype(jnp.bfloat16)
    out_bf = out_embed.astype(jnp.bfloat16)
    means = _ctx_means(context_words.reshape(-1), in_bf, B, CTX, D)
    pos_score, neg_score = _scores(
        center_word, neg_words.reshape(-1), means, out_bf, B, NEG, D)
    return _loss_from_scores(pos_score, neg_score, B)


# speedup vs baseline: 1.7792x; 1.7792x over previous
import functools

import jax
import jax.numpy as jnp
from jax import lax
from jax.experimental import pallas as pl
from jax.experimental.pallas import tpu as pltpu
from jax.experimental.pallas import tpu_sc as plsc

NC, NS = 2, 16
NW = NC * NS
LANES = 16
C = 16

_SC_PARAMS = pltpu.CompilerParams(needs_layout_passes=False)


def _pack_rows_tc(table):
    V, D = table.shape
    CH = 4096

    def body(t_ref, o_ref):
        x = t_ref[...]
        xt = x.T
        o_ref[...] = jnp.concatenate([xt, jnp.zeros_like(xt)], axis=1)

    return pl.pallas_call(
        body,
        grid=(pl.cdiv(V, CH),),
        in_specs=[pl.BlockSpec((D, CH), lambda j: (0, j))],
        out_specs=pl.BlockSpec((CH, 2 * D), lambda j: (j, 0)),
        out_shape=jax.ShapeDtypeStruct((V, 2 * D), jnp.float32),
    )(table.T)


def _ctx_means(ctx_flat, in_pk, B, CTX, D):
    BW = B // NW
    NIT = BW // C
    NKC = D // LANES
    nctx = C * CTX
    mesh = plsc.VectorSubcoreMesh(core_axis_name="c", subcore_axis_name="s")

    @functools.partial(
        pl.kernel,
        out_type=jax.ShapeDtypeStruct((B * D,), jnp.float32),
        mesh=mesh,
        compiler_params=_SC_PARAMS,
        scratch_types=[
            pltpu.VMEM((BW * CTX,), jnp.int32),
            pltpu.VMEM((nctx, 2 * D), jnp.float32),
            pltpu.VMEM((nctx, 2 * D), jnp.float32),
            pltpu.VMEM((BW * D,), jnp.float32),
            pltpu.SemaphoreType.DMA,
            pltpu.SemaphoreType.DMA,
        ],
    )
    def mean_kernel(ctx_hbm, ine_hbm, mean_o_hbm,
                    ctx_idx, rows0, rows1, mean_buf, sem0, sem1):
        wid = lax.axis_index("s") * NC + lax.axis_index("c")
        pltpu.sync_copy(ctx_hbm.at[pl.ds(wid * BW * CTX, BW * CTX)], ctx_idx)

        def fire(i, buf, sem):
            for h in range(2):
                pltpu.async_copy(
                    ine_hbm.at[ctx_idx.at[pl.ds(i * nctx + h * (nctx // 2),
                                                nctx // 2)]],
                    buf.at[pl.ds(h * (nctx // 2), nctx // 2)], sem)

        def drain(i, buf, sem):
            for h in range(2):
                pltpu.make_async_copy(
                    ine_hbm.at[ctx_idx.at[pl.ds(i * nctx + h * (nctx // 2),
                                                nctx // 2)]],
                    buf.at[pl.ds(h * (nctx // 2), nctx // 2)], sem).wait()

        def compute(i, rows):
            for b in range(C):
                m = [rows[b * CTX, pl.ds(k * LANES, LANES)]
                     for k in range(NKC)]
                for c in range(1, CTX):
                    r = b * CTX + c
                    m = [mk + rows[r, pl.ds(k * LANES, LANES)]
                         for k, mk in enumerate(m)]
                base = (i * C + b) * D
                for k in range(NKC):
                    mean_buf[pl.ds(base + k * LANES, LANES)] = (
                        m[k] * (1.0 / CTX))

        fire(0, rows0, sem0)

        def body(j, carry):
            i0 = 2 * j
            fire(i0 + 1, rows1, sem1)
            drain(i0, rows0, sem0)
            compute(i0, rows0)
            fire(jnp.minimum(i0 + 2, NIT - 1), rows0, sem0)
            drain(i0 + 1, rows1, sem1)
            compute(i0 + 1, rows1)
            return carry

        lax.fori_loop(0, NIT // 2, body, 0)
        drain(NIT - 1, rows0, sem0)
        pltpu.sync_copy(mean_buf, mean_o_hbm.at[pl.ds(wid * BW * D, BW * D)])

    return mean_kernel(ctx_flat, in_pk)


def _scores(cen, neg_flat, means, out_pk, B, NEG, D):
    BW = B // NW
    NIT = BW // C
    NKC = D // LANES
    nneg = C * NEG
    mesh = plsc.VectorSubcoreMesh(core_axis_name="c", subcore_axis_name="s")

    @functools.partial(
        pl.kernel,
        out_type=(
            jax.ShapeDtypeStruct((B,), jnp.float32),
            jax.ShapeDtypeStruct((B * NEG,), jnp.float32),
        ),
        mesh=mesh,
        compiler_params=_SC_PARAMS,
        scratch_types=[
            pltpu.VMEM((BW,), jnp.int32),
            pltpu.VMEM((BW * NEG,), jnp.int32),
            pltpu.VMEM((BW * D,), jnp.float32),
            pltpu.VMEM((C, 2 * D), jnp.float32),
            pltpu.VMEM((nneg, 2 * D), jnp.float32),
            pltpu.VMEM((C * LANES,), jnp.float32),
            pltpu.VMEM((nneg * LANES,), jnp.float32),
            pltpu.VMEM((BW,), jnp.float32),
            pltpu.VMEM((BW * NEG,), jnp.float32),
            pltpu.SemaphoreType.DMA,
        ],
    )
    def score_kernel(cen_hbm, neg_hbm, mean_hbm, oute_hbm,
                     pos_o_hbm, neg_o_hbm,
                     cen_idx, neg_idx, mean_v, pos_rows, neg_rows,
                     stage_pos, stage_neg, pos_buf, neg_buf, sem):
        wid = lax.axis_index("s") * NC + lax.axis_index("c")
        pltpu.sync_copy(cen_hbm.at[pl.ds(wid * BW, BW)], cen_idx)
        pltpu.sync_copy(neg_hbm.at[pl.ds(wid * BW * NEG, BW * NEG)], neg_idx)
        pltpu.sync_copy(mean_hbm.at[pl.ds(wid * BW * D, BW * D)], mean_v)

        def colsum(stage, r0):
            base = lax.iota(jnp.int32, 16) * LANES + (r0 * LANES)
            acc = plsc.load_gather(stage, [base])
            for c in range(1, LANES):
                acc = acc + plsc.load_gather(stage, [base + c])
            return acc

        def body(i, carry):
            dmas = [pltpu.async_copy(
                oute_hbm.at[cen_idx.at[pl.ds(i * C, C)]], pos_rows, sem)]
            for h in range(4):
                dmas.append(pltpu.async_copy(
                    oute_hbm.at[neg_idx.at[pl.ds(i * nneg + h * (nneg // 4),
                                                 nneg // 4)]],
                    neg_rows.at[pl.ds(h * (nneg // 4), nneg // 4)], sem))
            for d in dmas:
                d.wait()

            for b in range(C):
                base = (i * C + b) * D
                m = [mean_v[pl.ds(base + k * LANES, LANES)]
                     for k in range(NKC)]

                p = m[0] * pos_rows[b, pl.ds(0, LANES)]
                for k in range(1, NKC):
                    p = p + m[k] * pos_rows[b, pl.ds(k * LANES, LANES)]
                stage_pos[pl.ds(b * LANES, LANES)] = p

                for n in range(NEG):
                    row = b * NEG + n
                    q = m[0] * neg_rows[row, pl.ds(0, LANES)]
                    for k in range(1, NKC):
                        q = q + m[k] * neg_rows[row, pl.ds(k * LANES, LANES)]
                    stage_neg[pl.ds(row * LANES, LANES)] = q

            pos_buf[pl.ds(i * C, C)] = colsum(stage_pos, 0)
            for g in range(nneg // LANES):
                neg_buf[pl.ds(i * nneg + g * LANES, LANES)] = (
                    colsum(stage_neg, g * LANES))
            return carry

        lax.fori_loop(0, NIT, body, 0)
        pltpu.sync_copy(pos_buf, pos_o_hbm.at[pl.ds(wid * BW, BW)])
        pltpu.sync_copy(neg_buf, neg_o_hbm.at[pl.ds(wid * BW * NEG, BW * NEG)])

    return score_kernel(cen, neg_flat, means, out_pk)


def _loss_from_scores(pos_score, neg_score_flat, B):
    pos2 = pos_score.reshape(-1, 128)
    neg2 = neg_score_flat.reshape(-1, 128)

    def body(p_ref, n_ref, o_ref):
        def neg_softplus(x):
            return jnp.minimum(x, 0.0) - jnp.log(1.0 + jnp.exp(-jnp.abs(x)))

        total = jnp.sum(neg_softplus(p_ref[...]))
        total = total + jnp.sum(neg_softplus(-n_ref[...]))
        o_ref[0, 0] = -total / B

    out = pl.pallas_call(
        body,
        out_shape=jax.ShapeDtypeStruct((1, 1), jnp.float32),
        out_specs=pl.BlockSpec(memory_space=pltpu.SMEM),
    )(pos2, neg2)
    return out[0, 0]


def kernel(context_words, center_word, neg_words, in_embed, out_embed):
    B, CTX = context_words.shape
    NEG = neg_words.shape[1]
    D = in_embed.shape[1]
    in_pk = _pack_rows_tc(in_embed)
    out_pk = _pack_rows_tc(out_embed)
    means = _ctx_means(context_words.reshape(-1), in_pk, B, CTX, D)
    pos_score, neg_score = _scores(
        center_word, neg_words.reshape(-1), means, out_pk, B, NEG, D)
    return _loss_from_scores(pos_score, neg_score, B)
